# baseline (speedup 1.0000x reference)
import functools
import jax
import jax.numpy as jnp
from jax import lax
from jax.experimental import pallas as pl
from jax.experimental.pallas import tpu as pltpu
from jax.experimental.pallas import tpu_sc as plsc

VOCAB = 1000000
D = 128
MAX_SEQ = 512
B = 1024
S = 512
N = B * S
EPS = 1e-7

NC = 2
NS = 16
NW = NC * NS
N_PER_W = N // NW
CH = 128
NCH = N_PER_W // CH
NBUF = 4
U = 16
NV = D // 16


def _rsqrt(x):
    i = lax.bitcast_convert_type(x, jnp.int32)
    i = jnp.int32(0x5F3759DF) - lax.shift_right_logical(i, 1)
    y = lax.bitcast_convert_type(i, jnp.float32)
    y = y * (1.5 - (0.5 * x) * (y * y))
    return y


_GATHER_DNUMS = lax.GatherDimensionNumbers(
    offset_dims=(), collapsed_slice_dims=(0,), start_index_map=(0,))


def _permute(v, p):
    return lax.gather(v, p[:, None], _GATHER_DNUMS, slice_sizes=(1,),
                      mode=lax.GatherScatterMode.PROMISE_IN_BOUNDS)


def _lane_sum(v):
    for m in (1, 2, 4, 8):
        p = lax.iota(jnp.int32, 16) ^ m
        v = v + _permute(v, p)
    return v


def _tree8(x):
    return ((x[0] + x[1]) + (x[2] + x[3])) + ((x[4] + x[5]) + (x[6] + x[7]))


_mesh = plsc.VectorSubcoreMesh(core_axis_name="c", subcore_axis_name="s")


@functools.partial(
    pl.kernel,
    out_type=[
        jax.ShapeDtypeStruct((N, D), jnp.float32),
        jax.ShapeDtypeStruct((N, D), jnp.float32),
    ],
    mesh=_mesh,
    compiler_params=pltpu.CompilerParams(needs_layout_passes=False),
    scratch_types=[
        pltpu.VMEM((NCH, CH), jnp.int32),
        pltpu.VMEM_SHARED((S, D), jnp.float32),
        pltpu.VMEM((CH, D), jnp.float32),
        pltpu.VMEM((CH, D), jnp.float32),
        pltpu.VMEM((CH, D), jnp.float32),
        pltpu.VMEM((CH, D), jnp.float32),
        pltpu.VMEM((D,), jnp.float32),
        pltpu.VMEM((D,), jnp.float32),
        pltpu.SemaphoreType.DMA,
        pltpu.SemaphoreType.DMA,
        pltpu.SemaphoreType.DMA,
        pltpu.SemaphoreType.DMA,
        pltpu.SemaphoreType.DMA,
        pltpu.SemaphoreType.DMA,
        pltpu.SemaphoreType.DMA,
        pltpu.SemaphoreType.DMA,
        pltpu.SemaphoreType.DMA,
    ],
)
def _sc_embed(idx_hbm, table_hbm, pos_hbm, gamma_hbm, beta_hbm,
              out1_hbm, out2_hbm,
              idx_v, pos_v, buf0, buf1, buf2, buf3, gam_v, bet_v,
              g0, g1, g2, g3, s0, s1, s2, s3, pos_sem):
    bufs = [buf0, buf1, buf2, buf3]
    gsems = [g0, g1, g2, g3]
    ssems = [s0, s1, s2, s3]

    wid = lax.axis_index("s") * NC + lax.axis_index("c")
    base = wid * N_PER_W

    pltpu.sync_copy(idx_hbm.at[wid], idx_v)

    def ln_chunk(buf):
        def rows_body(t, _):
            r0 = t * U
            for u in range(U):
                r = r0 + u
                x = [buf[r, pl.ds(16 * j, 16)] for j in range(NV)]
                s = _tree8(x)
                q = _tree8([xi * xi for xi in x])
                mean = jnp.sum(s) * (1.0 / D)
                var = jnp.sum(q) * (1.0 / D) - mean * mean
                rstd = _rsqrt(var + EPS)
                mr = mean * rstd
                for j in range(NV):
                    t = x[j] * rstd - mr
                    buf[r, pl.ds(16 * j, 16)] = t * gvs[j] + bvs[j]
            return 0
        lax.fori_loop(0, CH // U, rows_body, 0)

    def wait_store(k):
        pltpu.make_async_copy(bufs[k], out1_hbm.at[pl.ds(0, CH)],
                              ssems[k]).wait()

    def wait_pos():
        pltpu.make_async_copy(pos_v.at[pl.ds(0, CH)],
                              out2_hbm.at[pl.ds(0, CH)], pos_sem).wait()

    def issue_gather(g, k):
        pltpu.async_copy(table_hbm.at[idx_v.at[g]], bufs[k], gsems[k])

    def wait_gather(k):
        pltpu.make_async_copy(table_hbm.at[pl.ds(0, CH)], bufs[k],
                              gsems[k]).wait()

    issue_gather(0, 0)
    issue_gather(1, 1)

    pltpu.sync_copy(gamma_hbm, gam_v)
    pltpu.sync_copy(beta_hbm, bet_v)

    @pl.when(lax.axis_index("s") == 0)
    def _():
        pltpu.sync_copy(pos_hbm, pos_v)

    plsc.subcore_barrier()

    gvs = [gam_v[pl.ds(16 * j, 16)] for j in range(NV)]
    bvs = [bet_v[pl.ds(16 * j, 16)] for j in range(NV)]

    def quad_body(i, _):
        for k in range(NBUF):
            g = NBUF * i + k
            wait_gather(k)
            ln_chunk(bufs[k])
            off = g * CH
            pltpu.async_copy(bufs[k], out1_hbm.at[pl.ds(base + off, CH)],
                             ssems[k])
            pltpu.async_copy(pos_v.at[pl.ds(off % S, CH)],
                             out2_hbm.at[pl.ds(base + off, CH)], pos_sem)

            @pl.when(g >= 1)
            def _():
                wait_pos()

            @pl.when(g + 2 < NCH)
            def _():
                kn = (k + 2) % NBUF

                @pl.when(g >= 2)
                def _():
                    wait_store(kn)

                issue_gather(g + 2, kn)
        return 0

    lax.fori_loop(0, NCH // NBUF, quad_body, 0)

    for k in range(NBUF):
        wait_store(k)
    wait_pos()


def kernel(inputs, word_table, pos_table, ln_gamma, ln_beta):
    idx = inputs.reshape(NW, NCH, CH).astype(jnp.int32)
    out1, out2 = _sc_embed(idx, word_table, pos_table, ln_gamma, ln_beta)
    return out1.reshape(B, S, D), out2.reshape(B, S, D)


# speedup vs baseline: 1.1688x; 1.1688x over previous
import functools
import jax
import jax.numpy as jnp
from jax import lax
from jax.experimental import pallas as pl
from jax.experimental.pallas import tpu as pltpu
from jax.experimental.pallas import tpu_sc as plsc

VOCAB = 1000000
D = 128
MAX_SEQ = 512
B = 1024
S = 512
N = B * S
EPS = 1e-7

NC = 2
NS = 16
NW = NC * NS
N_PER_W = N // NW
CH = 128
NCH = N_PER_W // CH
NBUF = 4
U = 8
NV = D // 16


def _rsqrt(x):
    i = lax.bitcast_convert_type(x, jnp.int32)
    i = jnp.int32(0x5F3759DF) - lax.shift_right_logical(i, 1)
    y = lax.bitcast_convert_type(i, jnp.float32)
    y = y * (1.5 - (0.5 * x) * (y * y))
    return y


_GATHER_DNUMS = lax.GatherDimensionNumbers(
    offset_dims=(), collapsed_slice_dims=(0,), start_index_map=(0,))


def _permute(v, p):
    return lax.gather(v, p[:, None], _GATHER_DNUMS, slice_sizes=(1,),
                      mode=lax.GatherScatterMode.PROMISE_IN_BOUNDS)


def _lane_sum(v):
    for m in (1, 2, 4, 8):
        p = lax.iota(jnp.int32, 16) ^ m
        v = v + _permute(v, p)
    return v


def _tree8(x):
    return ((x[0] + x[1]) + (x[2] + x[3])) + ((x[4] + x[5]) + (x[6] + x[7]))


_mesh = plsc.VectorSubcoreMesh(core_axis_name="c", subcore_axis_name="s")


@functools.partial(
    pl.kernel,
    out_type=[
        jax.ShapeDtypeStruct((N, D), jnp.float32),
        jax.ShapeDtypeStruct((N, D), jnp.float32),
    ],
    mesh=_mesh,
    compiler_params=pltpu.CompilerParams(needs_layout_passes=False),
    scratch_types=[
        pltpu.VMEM((NCH, CH), jnp.int32),
        pltpu.VMEM_SHARED((S, D), jnp.float32),
        pltpu.VMEM((CH, D), jnp.float32),
        pltpu.VMEM((CH, D), jnp.float32),
        pltpu.VMEM((CH, D), jnp.float32),
        pltpu.VMEM((CH, D), jnp.float32),
        pltpu.VMEM((D,), jnp.float32),
        pltpu.VMEM((D,), jnp.float32),
        pltpu.SemaphoreType.DMA,
        pltpu.SemaphoreType.DMA,
        pltpu.SemaphoreType.DMA,
        pltpu.SemaphoreType.DMA,
        pltpu.SemaphoreType.DMA,
        pltpu.SemaphoreType.DMA,
        pltpu.SemaphoreType.DMA,
        pltpu.SemaphoreType.DMA,
        pltpu.SemaphoreType.DMA,
    ],
)
def _sc_embed(idx_hbm, table_hbm, pos_hbm, gamma_hbm, beta_hbm,
              out1_hbm, out2_hbm,
              idx_v, pos_v, buf0, buf1, buf2, buf3, gam_v, bet_v,
              g0, g1, g2, g3, s0, s1, s2, s3, pos_sem):
    bufs = [buf0, buf1, buf2, buf3]
    gsems = [g0, g1, g2, g3]
    ssems = [s0, s1, s2, s3]

    wid = lax.axis_index("s") * NC + lax.axis_index("c")
    base = wid * N_PER_W

    pltpu.sync_copy(idx_hbm.at[wid], idx_v)

    def ln_chunk(buf):
        def rows_body(t, _):
            r0 = t * U
            for u in range(U):
                r = r0 + u
                x = [buf[r, pl.ds(16 * j, 16)] for j in range(NV)]
                s = _tree8(x)
                q = _tree8([xi * xi for xi in x])
                mean = jnp.sum(s) * (1.0 / D)
                var = jnp.sum(q) * (1.0 / D) - mean * mean
                rstd = _rsqrt(var + EPS)
                mr = mean * rstd
                for j in range(NV):
                    t = x[j] * rstd - mr
                    buf[r, pl.ds(16 * j, 16)] = t * gvs[j] + bvs[j]
            return 0
        lax.fori_loop(0, CH // U, rows_body, 0)

    def wait_store(k):
        pltpu.make_async_copy(bufs[k], out1_hbm.at[pl.ds(0, CH)],
                              ssems[k]).wait()

    def wait_pos():
        pltpu.make_async_copy(pos_v.at[pl.ds(0, CH)],
                              out2_hbm.at[pl.ds(0, CH)], pos_sem).wait()

    def issue_gather(g, k):
        pltpu.async_copy(table_hbm.at[idx_v.at[g]], bufs[k], gsems[k])

    def wait_gather(k):
        pltpu.make_async_copy(table_hbm.at[pl.ds(0, CH)], bufs[k],
                              gsems[k]).wait()

    issue_gather(0, 0)
    issue_gather(1, 1)

    pltpu.sync_copy(gamma_hbm, gam_v)
    pltpu.sync_copy(beta_hbm, bet_v)

    @pl.when(lax.axis_index("s") == 0)
    def _():
        pltpu.sync_copy(pos_hbm, pos_v)

    plsc.subcore_barrier()

    gvs = [gam_v[pl.ds(16 * j, 16)] for j in range(NV)]
    bvs = [bet_v[pl.ds(16 * j, 16)] for j in range(NV)]

    def quad_body(i, _):
        for k in range(NBUF):
            g = NBUF * i + k
            wait_gather(k)
            ln_chunk(bufs[k])
            off = g * CH
            pltpu.async_copy(bufs[k], out1_hbm.at[pl.ds(base + off, CH)],
                             ssems[k])
            pltpu.async_copy(pos_v.at[pl.ds(off % S, CH)],
                             out2_hbm.at[pl.ds(base + off, CH)], pos_sem)

            @pl.when(g >= 1)
            def _():
                wait_pos()

            @pl.when(g + 2 < NCH)
            def _():
                kn = (k + 2) % NBUF

                @pl.when(g >= 2)
                def _():
                    wait_store(kn)

                issue_gather(g + 2, kn)
        return 0

    lax.fori_loop(0, NCH // NBUF, quad_body, 0)

    for k in range(NBUF):
        wait_store(k)
    wait_pos()


def kernel(inputs, word_table, pos_table, ln_gamma, ln_beta):
    idx = inputs.reshape(NW, NCH, CH).astype(jnp.int32)
    out1, out2 = _sc_embed(idx, word_table, pos_table, ln_gamma, ln_beta)
    return out1.reshape(B, S, D), out2.reshape(B, S, D)
